# baseline (speedup 1.0000x reference)
import jax
import jax.numpy as jnp
from jax.experimental import pallas as pl
from jax.experimental.pallas import tpu as pltpu

TOKENS = 16384
DIM = 2048
N_EXPERTS = 16
K = 2
BT = 512


def _router_block(x_ref, w_ref, gatesT_ref, valsT_ref, idxT_ref):
    logitsT = jax.lax.dot_general(
        w_ref[...], x_ref[...], (((1,), (1,)), ((), ())),
        preferred_element_type=jnp.float32,
    )
    m = jnp.max(logitsT, axis=0, keepdims=True)
    e = jnp.exp(logitsT - m)
    s = jnp.sum(e, axis=0, keepdims=True)
    gatesT = e / s
    gatesT_ref[...] = gatesT
    iota = jax.lax.broadcasted_iota(jnp.int32, gatesT.shape, 0)
    v1 = jnp.max(gatesT, axis=0, keepdims=True)
    i1 = jnp.min(jnp.where(gatesT == v1, iota, N_EXPERTS), axis=0, keepdims=True)
    masked = jnp.where(iota == i1, -jnp.inf, gatesT)
    v2 = jnp.max(masked, axis=0, keepdims=True)
    i2 = jnp.min(jnp.where(masked == v2, iota, N_EXPERTS), axis=0, keepdims=True)
    valsT_ref[...] = jnp.concatenate([v1, v2], axis=0)
    idxT_ref[...] = jnp.concatenate([i1, i2], axis=0)


@jax.jit
def kernel(x, W):
    grid = (TOKENS // BT,)
    gatesT, valsT, idxT = pl.pallas_call(
        _router_block,
        grid=grid,
        in_specs=[
            pl.BlockSpec((BT, DIM), lambda i: (i, 0)),
            pl.BlockSpec((N_EXPERTS, DIM), lambda i: (0, 0)),
        ],
        out_specs=[
            pl.BlockSpec((N_EXPERTS, BT), lambda i: (0, i)),
            pl.BlockSpec((K, BT), lambda i: (0, i)),
            pl.BlockSpec((K, BT), lambda i: (0, i)),
        ],
        out_shape=[
            jax.ShapeDtypeStruct((N_EXPERTS, TOKENS), jnp.float32),
            jax.ShapeDtypeStruct((K, TOKENS), jnp.float32),
            jax.ShapeDtypeStruct((K, TOKENS), jnp.int32),
        ],
    )(x, W)
    return (gatesT.T, valsT.T, idxT.T)


# speedup vs baseline: 1.1402x; 1.1402x over previous
import jax
import jax.numpy as jnp
from jax.experimental import pallas as pl
from jax.experimental.pallas import tpu as pltpu

TOKENS = 16384
DIM = 2048
N_EXPERTS = 16
K = 2
BT = 2048


def _router_block(x_ref, w_ref, gatesT_ref, valsT_ref, idxT_ref):
    logitsT = jax.lax.dot_general(
        w_ref[...], x_ref[...], (((1,), (1,)), ((), ())),
        preferred_element_type=jnp.float32,
    )
    m = jnp.max(logitsT, axis=0, keepdims=True)
    e = jnp.exp(logitsT - m)
    s = jnp.sum(e, axis=0, keepdims=True)
    gatesT = e / s
    gatesT_ref[...] = gatesT
    iota = jax.lax.broadcasted_iota(jnp.int32, gatesT.shape, 0)
    v1 = jnp.max(gatesT, axis=0, keepdims=True)
    i1 = jnp.min(jnp.where(gatesT == v1, iota, N_EXPERTS), axis=0, keepdims=True)
    masked = jnp.where(iota == i1, -jnp.inf, gatesT)
    v2 = jnp.max(masked, axis=0, keepdims=True)
    i2 = jnp.min(jnp.where(masked == v2, iota, N_EXPERTS), axis=0, keepdims=True)
    valsT_ref[...] = jnp.concatenate([v1, v2], axis=0)
    idxT_ref[...] = jnp.concatenate([i1, i2], axis=0)


@jax.jit
def kernel(x, W):
    grid = (TOKENS // BT,)
    gatesT, valsT, idxT = pl.pallas_call(
        _router_block,
        grid=grid,
        in_specs=[
            pl.BlockSpec((BT, DIM), lambda i: (i, 0)),
            pl.BlockSpec((N_EXPERTS, DIM), lambda i: (0, 0)),
        ],
        out_specs=[
            pl.BlockSpec((N_EXPERTS, BT), lambda i: (0, i)),
            pl.BlockSpec((K, BT), lambda i: (0, i)),
            pl.BlockSpec((K, BT), lambda i: (0, i)),
        ],
        out_shape=[
            jax.ShapeDtypeStruct((N_EXPERTS, TOKENS), jnp.float32),
            jax.ShapeDtypeStruct((K, TOKENS), jnp.float32),
            jax.ShapeDtypeStruct((K, TOKENS), jnp.int32),
        ],
    )(x, W)
    return (gatesT.T, valsT.T, idxT.T)


# speedup vs baseline: 1.1979x; 1.0506x over previous
import jax
import jax.numpy as jnp
from jax.experimental import pallas as pl
from jax.experimental.pallas import tpu as pltpu

TOKENS = 16384
DIM = 2048
N_EXPERTS = 16
K = 2
BT = 1024


def _router_block(x_ref, w_ref, gatesT_ref, valsT_ref, idxT_ref):
    logitsT = jax.lax.dot_general(
        w_ref[...], x_ref[...], (((1,), (1,)), ((), ())),
        preferred_element_type=jnp.float32,
    )
    m = jnp.max(logitsT, axis=0, keepdims=True)
    e = jnp.exp(logitsT - m)
    s = jnp.sum(e, axis=0, keepdims=True)
    gatesT = e / s
    gatesT_ref[...] = gatesT
    iota = jax.lax.broadcasted_iota(jnp.int32, gatesT.shape, 0)
    v1 = jnp.max(gatesT, axis=0, keepdims=True)
    i1 = jnp.min(jnp.where(gatesT == v1, iota, N_EXPERTS), axis=0, keepdims=True)
    masked = jnp.where(iota == i1, -jnp.inf, gatesT)
    v2 = jnp.max(masked, axis=0, keepdims=True)
    i2 = jnp.min(jnp.where(masked == v2, iota, N_EXPERTS), axis=0, keepdims=True)
    valsT_ref[...] = jnp.concatenate([v1, v2], axis=0)
    idxT_ref[...] = jnp.concatenate([i1, i2], axis=0)


@jax.jit
def kernel(x, W):
    grid = (TOKENS // BT,)
    gatesT, valsT, idxT = pl.pallas_call(
        _router_block,
        grid=grid,
        in_specs=[
            pl.BlockSpec((BT, DIM), lambda i: (i, 0)),
            pl.BlockSpec((N_EXPERTS, DIM), lambda i: (0, 0)),
        ],
        out_specs=[
            pl.BlockSpec((N_EXPERTS, BT), lambda i: (0, i)),
            pl.BlockSpec((K, BT), lambda i: (0, i)),
            pl.BlockSpec((K, BT), lambda i: (0, i)),
        ],
        out_shape=[
            jax.ShapeDtypeStruct((N_EXPERTS, TOKENS), jnp.float32),
            jax.ShapeDtypeStruct((K, TOKENS), jnp.float32),
            jax.ShapeDtypeStruct((K, TOKENS), jnp.int32),
        ],
    )(x, W)
    return (gatesT.T, valsT.T, idxT.T)
